# baseline (speedup 1.0000x reference)
import functools

import jax
import jax.numpy as jnp
from jax import lax
from jax.experimental import pallas as pl
from jax.experimental.pallas import tpu as pltpu

NUM_CLASSES = 100
PRE_OUT = 512
N_PART = 8
PART_LAYER = 512
NUM_DOM = 4
HIDDEN = PART_LAYER // N_PART
TAU = 0.1
EPS = 1e-5
F32 = jnp.float32
BF16 = jnp.bfloat16
HI = lax.Precision.HIGHEST


def _bdot(a, w_bf16):
    return lax.dot_general(a.astype(BF16), w_bf16, (((1,), (0,)), ((), ())),
                           preferred_element_type=F32)


def _fold_affine(st_ref, g_ref, b_ref, n, cin):
    st = st_ref[...]
    L = st.shape[1]
    rowc = lax.broadcasted_iota(jnp.int32, (L, L), 0) % cin
    colc = lax.broadcasted_iota(jnp.int32, (L, L), 1) % cin
    t = (rowc == colc).astype(F32)
    folded = lax.dot_general(st, t, (((1,), (0,)), ((), ())),
                             preferred_element_type=F32, precision=HI)
    mean = folded[0:1, :] / n
    var = folded[1:2, :] / n - mean * mean
    scale = g_ref[...] * lax.rsqrt(var + EPS)
    shift = b_ref[...] - mean * scale
    return scale, shift


def _accum_stats(st_ref, y2d):
    s0 = jnp.sum(y2d, axis=0, keepdims=True)
    s1 = jnp.sum(y2d * y2d, axis=0, keepdims=True)

    @pl.when(pl.program_id(0) == 0)
    def _():
        st_ref[...] = jnp.zeros_like(st_ref)

    st_ref[...] += jnp.concatenate([s0, s1], axis=0)


def _k_conv1(x_ref, w_ref, b_ref, y_ref, st_ref, *, bt):
    x = x_ref[...].reshape(bt, 32, 96)
    xp = jnp.pad(x, ((0, 0), (1, 1), (3, 3)))
    y = b_ref[...]
    for i in range(3):
        y = y + _bdot(xp[:, i:i + 32, :].reshape(bt * 32, 102), w_ref[i])
    _accum_stats(st_ref, y)
    y_ref[...] = y


def _k_norm_pool_conv(y_ref, st_ref, g_ref, b_ref, w_ref, cb_ref,
                      o_ref, st2_ref, *, bt, n, h, cin):
    scale, shift = _fold_affine(st_ref, g_ref, b_ref, n, cin)
    y = y_ref[...]
    yn = jnp.maximum(y * scale + shift, 0.0)
    hh = h // 2
    yn = yn.reshape(bt, hh, 2, h * cin).max(axis=2).reshape(bt * hh, h * cin)
    rolled = jnp.concatenate([yn[:, cin:], yn[:, :cin]], axis=1)
    q = jnp.maximum(yn, rolled)
    qp = jnp.pad(q.reshape(bt, hh, h * cin),
                 ((0, 0), (1, 1), (2 * cin, 2 * cin)))
    z = cb_ref[...]
    for i in range(3):
        z = z + _bdot(qp[:, i:i + hh, :].reshape(bt * hh, (h + 4) * cin),
                      w_ref[i])
    _accum_stats(st2_ref, z)
    o_ref[...] = z


def _k_pre(y_ref, st_ref, g_ref, b_ref, w_ref, pb_ref, lg_ref, lb_ref,
           f_ref, *, bt, n):
    scale, shift = _fold_affine(st_ref, g_ref, b_ref, n, 64)
    y = y_ref[...]
    yn = jnp.maximum(y * scale + shift, 0.0).reshape(bt, 8, 512)
    f = pb_ref[...]
    for hrow in range(8):
        f = f + _bdot(yn[:, hrow, :], w_ref[hrow])
    mu = jnp.mean(f, axis=1, keepdims=True)
    d = f - mu
    v = jnp.mean(d * d, axis=1, keepdims=True)
    f = d * lax.rsqrt(v + EPS) * lg_ref[...] + lb_ref[...]
    f_ref[...] = jnp.maximum(f, 0.0)


def _k_disc(f_ref, w_ref, b_ref, o_ref, st_ref):
    f = f_ref[...]
    z = _bdot(f, w_ref[...]) + b_ref[...]
    _accum_stats(st_ref, z[:, :PART_LAYER])
    o_ref[...] = z


def _k_route(dz_ref, hh_ref, st_ref, g_ref, b_ref, wh_ref, bh_ref, u_ref,
             dom_ref, idx_ref, probs_ref, s1_ref, s2_ref, cnt_ref, *, n):
    st = st_ref[...]
    mean = st[0:1, :] / n
    var = st[1:2, :] / n - mean * mean
    inv = lax.rsqrt(var + EPS)
    scale = g_ref[...] * inv
    shift = b_ref[...] - mean * scale
    d = jnp.maximum(dz_ref[...] * scale + shift, 0.0)
    heads = _bdot(d, wh_ref[...]) + bh_ref[...]
    u = u_ref[...]
    z = heads[:, NUM_DOM:NUM_DOM + N_PART] - jnp.log(-jnp.log(u))
    zmax = jnp.max(z, axis=1, keepdims=True)
    iot = lax.broadcasted_iota(jnp.int32, (1, N_PART), 1)
    cand = jnp.where(z >= zmax, iot, N_PART)
    idx2d = jnp.min(cand, axis=1, keepdims=True)
    onehot = (iot == idx2d).astype(F32)
    dom_ref[...] = heads
    idx_ref[...] = jnp.broadcast_to(idx2d, idx_ref.shape)
    probs_ref[...] = jnp.pad(onehot, ((0, 0), (0, 128 - N_PART)))
    hh = hh_ref[...]
    s1 = lax.dot_general(onehot, hh, (((0,), (0,)), ((), ())),
                         preferred_element_type=F32, precision=HI)
    s2 = lax.dot_general(onehot, hh * hh, (((0,), (0,)), ((), ())),
                         preferred_element_type=F32, precision=HI)
    cnt = jnp.sum(onehot, axis=0, keepdims=True)

    @pl.when(pl.program_id(0) == 0)
    def _():
        s1_ref[...] = jnp.zeros_like(s1_ref)
        s2_ref[...] = jnp.zeros_like(s2_ref)
        cnt_ref[...] = jnp.zeros_like(cnt_ref)

    s1_ref[...] += s1
    s2_ref[...] += s2
    cnt_ref[...] += jnp.broadcast_to(cnt.reshape(N_PART, 1), cnt_ref.shape)


def _k_part2(hh_ref, s1_ref, s2_ref, cnt_ref, pg_ref, pbb_ref, probs_ref,
             w2_ref, b2_ref, out_ref):
    cnt = jnp.maximum(cnt_ref[...], 1.0)
    mean = s1_ref[...] / cnt
    var = s2_ref[...] / cnt - mean * mean
    inv = lax.rsqrt(var + EPS)
    row = lax.broadcasted_iota(jnp.int32, (N_PART, PART_LAYER), 0)
    col = lax.broadcasted_iota(jnp.int32, (N_PART, PART_LAYER), 1) // HIDDEN
    sel = (row == col).astype(F32)
    scale_full = inv * pg_ref[...]
    scale = jnp.sum(scale_full * sel, axis=0, keepdims=True)
    shift = jnp.sum((pbb_ref[...] - mean * scale_full) * sel,
                    axis=0, keepdims=True)
    hn = jnp.maximum(hh_ref[...] * scale + shift, 0.0)
    probs = probs_ref[...]
    bt = hn.shape[0]
    acc = jnp.zeros((bt, 128), F32)
    for part in range(N_PART):
        blk = hn[:, part * HIDDEN:(part + 1) * HIDDEN]
        oo = _bdot(blk, w2_ref[part]) + b2_ref[part:part + 1, :]
        acc += probs[:, part:part + 1] * oo
    out_ref[...] = acc


def _band(conv_w, wdim, cin, cout):
    mats = []
    for i in range(3):
        m = jnp.zeros(((wdim + 2) * cin, wdim * cout), F32)
        for j in range(3):
            m = m + jnp.kron(jnp.eye(wdim + 2, wdim, -j, dtype=F32),
                             conv_w[:, :, i, j].T)
        mats.append(m)
    return jnp.stack(mats).astype(BF16)


def _band_pooled(conv_w, h, cin, cout):
    hh = h // 2
    mats = []
    for i in range(3):
        m = jnp.zeros(((h + 4) * cin, hh * cout), F32)
        for j in range(3):
            sel = jnp.zeros((h + 4, hh), F32).at[
                2 * jnp.arange(hh) + 2 * j, jnp.arange(hh)].set(1.0)
            m = m + jnp.kron(sel, conv_w[:, :, i, j].T)
        mats.append(m)
    return jnp.stack(mats).astype(BF16)


def kernel(input_data, params, u):
    p = params
    B = input_data.shape[0]
    x2d = jnp.transpose(input_data, (0, 2, 3, 1)).reshape(B * 32, 96)

    w1 = _band(p['conv1_w'], 32, 3, 16)
    b1 = jnp.tile(p['conv1_b'], 32).reshape(1, 512)
    w2 = _band_pooled(p['conv2_w'], 32, 16, 32)
    b2 = jnp.tile(p['conv2_b'], 16).reshape(1, 512)
    w3 = _band_pooled(p['conv3_w'], 16, 32, 64)
    b3 = jnp.tile(p['conv3_b'], 8).reshape(1, 512)
    g1t = jnp.tile(p['bn1_g'], 32).reshape(1, 512)
    b1t = jnp.tile(p['bn1_b'], 32).reshape(1, 512)
    g2t = jnp.tile(p['bn2_g'], 16).reshape(1, 512)
    b2t = jnp.tile(p['bn2_b'], 16).reshape(1, 512)
    g3t = jnp.tile(p['bn3_g'], 8).reshape(1, 512)
    b3t = jnp.tile(p['bn3_b'], 8).reshape(1, 512)
    w4 = p['pre_w'].reshape(PRE_OUT, 64, 8, 8).transpose(2, 3, 1, 0) \
        .reshape(8, 512, PRE_OUT).astype(BF16)
    wd = jnp.concatenate(
        [p['disc_w'].T, p['pw1'].reshape(PART_LAYER, PRE_OUT).T],
        axis=1).astype(BF16)
    bd = jnp.concatenate([p['disc_b'], p['pb1'].reshape(-1)]).reshape(1, -1)
    wh = jnp.pad(jnp.concatenate([p['dfc_w'].T, p['sw_w'].T], axis=1),
                 ((0, 0), (0, 128 - NUM_DOM - N_PART))).astype(BF16)
    bh = jnp.pad(jnp.concatenate([p['dfc_b'], p['sw_b']]),
                 (0, 128 - NUM_DOM - N_PART)).reshape(1, -1)
    w2p = jnp.pad(jnp.transpose(p['pw2'], (0, 2, 1)),
                  ((0, 0), (0, 0), (0, 128 - NUM_CLASSES))).astype(BF16)
    b2p = jnp.pad(p['pb2'], ((0, 0), (0, 128 - NUM_CLASSES)))
    pg = p['pbn_g'].reshape(1, PART_LAYER)
    pbb = jnp.broadcast_to(p['pbn_b'].reshape(-1)[None, :],
                           (N_PART, PART_LAYER))

    full = lambda s: pl.BlockSpec(s, lambda i: tuple(0 for _ in s))

    bt1 = min(128, B)
    y1, st1 = pl.pallas_call(
        functools.partial(_k_conv1, bt=bt1),
        grid=(B // bt1,),
        in_specs=[pl.BlockSpec((bt1 * 32, 96), lambda i: (i, 0)),
                  full((3, 102, 512)), full((1, 512))],
        out_specs=[pl.BlockSpec((bt1 * 32, 512), lambda i: (i, 0)),
                   full((2, 512))],
        out_shape=[jax.ShapeDtypeStruct((B * 32, 512), F32),
                   jax.ShapeDtypeStruct((2, 512), F32)],
    )(x2d, w1, b1)

    bt2 = min(128, B)
    y2, st2 = pl.pallas_call(
        functools.partial(_k_norm_pool_conv, bt=bt2, n=float(B * 1024),
                          h=32, cin=16),
        grid=(B // bt2,),
        in_specs=[pl.BlockSpec((bt2 * 32, 512), lambda i: (i, 0)),
                  full((2, 512)), full((1, 512)), full((1, 512)),
                  full((3, 576, 512)), full((1, 512))],
        out_specs=[pl.BlockSpec((bt2 * 16, 512), lambda i: (i, 0)),
                   full((2, 512))],
        out_shape=[jax.ShapeDtypeStruct((B * 16, 512), F32),
                   jax.ShapeDtypeStruct((2, 512), F32)],
    )(y1, st1, g1t, b1t, w2, b2)

    bt3 = min(128, B)
    y3, st3 = pl.pallas_call(
        functools.partial(_k_norm_pool_conv, bt=bt3, n=float(B * 256),
                          h=16, cin=32),
        grid=(B // bt3,),
        in_specs=[pl.BlockSpec((bt3 * 16, 512), lambda i: (i, 0)),
                  full((2, 512)), full((1, 512)), full((1, 512)),
                  full((3, 640, 512)), full((1, 512))],
        out_specs=[pl.BlockSpec((bt3 * 8, 512), lambda i: (i, 0)),
                   full((2, 512))],
        out_shape=[jax.ShapeDtypeStruct((B * 8, 512), F32),
                   jax.ShapeDtypeStruct((2, 512), F32)],
    )(y2, st2, g2t, b2t, w3, b3)

    bt4 = min(256, B)
    f = pl.pallas_call(
        functools.partial(_k_pre, bt=bt4, n=float(B * 64)),
        grid=(B // bt4,),
        in_specs=[pl.BlockSpec((bt4 * 8, 512), lambda i: (i, 0)),
                  full((2, 512)), full((1, 512)), full((1, 512)),
                  full((8, 512, PRE_OUT)), full((1, PRE_OUT)),
                  full((1, PRE_OUT)), full((1, PRE_OUT))],
        out_specs=pl.BlockSpec((bt4, PRE_OUT), lambda i: (i, 0)),
        out_shape=jax.ShapeDtypeStruct((B, PRE_OUT), F32),
    )(y3, st3, g3t, b3t,
      w4, p['pre_b'].reshape(1, -1), p['ln_g'].reshape(1, -1),
      p['ln_b'].reshape(1, -1))

    bt5 = min(512, B)
    dz, dst = pl.pallas_call(
        _k_disc,
        grid=(B // bt5,),
        in_specs=[pl.BlockSpec((bt5, PRE_OUT), lambda i: (i, 0)),
                  full((PRE_OUT, 1024)), full((1, 1024))],
        out_specs=[pl.BlockSpec((bt5, 1024), lambda i: (i, 0)),
                   full((2, PART_LAYER))],
        out_shape=[jax.ShapeDtypeStruct((B, 1024), F32),
                   jax.ShapeDtypeStruct((2, PART_LAYER), F32)],
    )(f, wd, bd)

    bt6 = min(512, B)
    dom, idxp, probsp, s1, s2, cnt = pl.pallas_call(
        functools.partial(_k_route, n=float(B)),
        grid=(B // bt6,),
        in_specs=[pl.BlockSpec((bt6, PART_LAYER), lambda i: (i, 0)),
                  pl.BlockSpec((bt6, PART_LAYER), lambda i: (i, 1)),
                  full((2, PART_LAYER)),
                  full((1, PART_LAYER)), full((1, PART_LAYER)),
                  full((PART_LAYER, 128)), full((1, 128)),
                  pl.BlockSpec((bt6, N_PART), lambda i: (i, 0))],
        out_specs=[pl.BlockSpec((bt6, 128), lambda i: (i, 0)),
                   pl.BlockSpec((bt6, 128), lambda i: (i, 0)),
                   pl.BlockSpec((bt6, 128), lambda i: (i, 0)),
                   full((N_PART, PART_LAYER)), full((N_PART, PART_LAYER)),
                   full((N_PART, PART_LAYER))],
        out_shape=[jax.ShapeDtypeStruct((B, 128), F32),
                   jax.ShapeDtypeStruct((B, 128), jnp.int32),
                   jax.ShapeDtypeStruct((B, 128), F32),
                   jax.ShapeDtypeStruct((N_PART, PART_LAYER), F32),
                   jax.ShapeDtypeStruct((N_PART, PART_LAYER), F32),
                   jax.ShapeDtypeStruct((N_PART, PART_LAYER), F32)],
    )(dz, dz, dst, p['dbn_g'].reshape(1, -1), p['dbn_b'].reshape(1, -1),
      wh, bh, u)

    bt7 = min(512, B)
    outp = pl.pallas_call(
        _k_part2,
        grid=(B // bt7,),
        in_specs=[pl.BlockSpec((bt7, PART_LAYER), lambda i: (i, 1)),
                  full((N_PART, PART_LAYER)), full((N_PART, PART_LAYER)),
                  full((N_PART, PART_LAYER)),
                  full((1, PART_LAYER)), full((N_PART, PART_LAYER)),
                  pl.BlockSpec((bt7, 128), lambda i: (i, 0)),
                  full((N_PART, HIDDEN, 128)), full((N_PART, 128))],
        out_specs=pl.BlockSpec((bt7, 128), lambda i: (i, 0)),
        out_shape=jax.ShapeDtypeStruct((B, 128), F32),
    )(dz, s1, s2, cnt, pg, pbb, probsp, w2p, b2p)

    out = outp[:, :NUM_CLASSES]
    domain_out = dom[:, :NUM_DOM]
    idx = idxp[:, 0]
    probs = probsp[:, :N_PART]
    return out, domain_out, idx, probs


# speedup vs baseline: 1.0048x; 1.0048x over previous
import functools

import jax
import jax.numpy as jnp
from jax import lax
from jax.experimental import pallas as pl
from jax.experimental.pallas import tpu as pltpu

NUM_CLASSES = 100
PRE_OUT = 512
N_PART = 8
PART_LAYER = 512
NUM_DOM = 4
HIDDEN = PART_LAYER // N_PART
TAU = 0.1
EPS = 1e-5
F32 = jnp.float32
BF16 = jnp.bfloat16
HI = lax.Precision.HIGHEST


def _bdot(a, w_bf16):
    return lax.dot_general(a.astype(BF16), w_bf16, (((1,), (0,)), ((), ())),
                           preferred_element_type=F32)


def _fold_affine(st_ref, g_ref, b_ref, n, cin):
    st = st_ref[...]
    L = st.shape[1]
    rowc = lax.broadcasted_iota(jnp.int32, (L, L), 0) % cin
    colc = lax.broadcasted_iota(jnp.int32, (L, L), 1) % cin
    t = (rowc == colc).astype(F32)
    folded = lax.dot_general(st, t, (((1,), (0,)), ((), ())),
                             preferred_element_type=F32, precision=HI)
    mean = folded[0:1, :] / n
    var = folded[1:2, :] / n - mean * mean
    scale = g_ref[...] * lax.rsqrt(var + EPS)
    shift = b_ref[...] - mean * scale
    return scale, shift


def _accum_stats(st_ref, y2d):
    s0 = jnp.sum(y2d, axis=0, keepdims=True)
    s1 = jnp.sum(y2d * y2d, axis=0, keepdims=True)

    @pl.when(pl.program_id(0) == 0)
    def _():
        st_ref[...] = jnp.zeros_like(st_ref)

    st_ref[...] += jnp.concatenate([s0, s1], axis=0)


def _k_conv1(x_ref, w_ref, b_ref, y_ref, st_ref, *, bt):
    x = x_ref[...].reshape(bt, 32, 96)
    xp = jnp.pad(x, ((0, 0), (1, 1), (3, 3)))
    y = b_ref[...]
    for i in range(3):
        y = y + _bdot(xp[:, i:i + 32, :].reshape(bt * 32, 102), w_ref[i])
    _accum_stats(st_ref, y)
    y_ref[...] = y


def _k_norm_pool_conv(y_ref, st_ref, g_ref, b_ref, w_ref, cb_ref,
                      o_ref, st2_ref, *, bt, n, h, cin):
    scale, shift = _fold_affine(st_ref, g_ref, b_ref, n, cin)
    y = y_ref[...]
    yn = jnp.maximum(y * scale + shift, 0.0)
    hh = h // 2
    yn = yn.reshape(bt, hh, 2, h * cin).max(axis=2).reshape(bt * hh, h * cin)
    rolled = jnp.concatenate([yn[:, cin:], yn[:, :cin]], axis=1)
    q = jnp.maximum(yn, rolled)
    qp = jnp.pad(q.reshape(bt, hh, h * cin),
                 ((0, 0), (1, 1), (2 * cin, 2 * cin)))
    z = cb_ref[...]
    for i in range(3):
        z = z + _bdot(qp[:, i:i + hh, :].reshape(bt * hh, (h + 4) * cin),
                      w_ref[i])
    _accum_stats(st2_ref, z)
    o_ref[...] = z


def _k_pre_disc(y_ref, st_ref, g_ref, b_ref, w_ref, pb_ref, lg_ref, lb_ref,
                wd_ref, bd_ref, o_ref, dst_ref, *, bt, n):
    scale, shift = _fold_affine(st_ref, g_ref, b_ref, n, 64)
    y = y_ref[...]
    yn = jnp.maximum(y * scale + shift, 0.0).reshape(bt, 8, 512)
    f = pb_ref[...]
    for hrow in range(8):
        f = f + _bdot(yn[:, hrow, :], w_ref[hrow])
    mu = jnp.mean(f, axis=1, keepdims=True)
    d = f - mu
    v = jnp.mean(d * d, axis=1, keepdims=True)
    f = d * lax.rsqrt(v + EPS) * lg_ref[...] + lb_ref[...]
    f = jnp.maximum(f, 0.0)
    z = _bdot(f, wd_ref[...]) + bd_ref[...]
    _accum_stats(dst_ref, z[:, :PART_LAYER])
    o_ref[...] = z


def _k_route(dz_ref, hh_ref, st_ref, g_ref, b_ref, wh_ref, bh_ref, u_ref,
             dom_ref, idx_ref, probs_ref, s1_ref, s2_ref, cnt_ref, *, n):
    st = st_ref[...]
    mean = st[0:1, :] / n
    var = st[1:2, :] / n - mean * mean
    inv = lax.rsqrt(var + EPS)
    scale = g_ref[...] * inv
    shift = b_ref[...] - mean * scale
    d = jnp.maximum(dz_ref[...] * scale + shift, 0.0)
    heads = _bdot(d, wh_ref[...]) + bh_ref[...]
    u = u_ref[...]
    z = heads[:, NUM_DOM:NUM_DOM + N_PART] - jnp.log(-jnp.log(u))
    zmax = jnp.max(z, axis=1, keepdims=True)
    iot = lax.broadcasted_iota(jnp.int32, (1, N_PART), 1)
    cand = jnp.where(z >= zmax, iot, N_PART)
    idx2d = jnp.min(cand, axis=1, keepdims=True)
    onehot = (iot == idx2d).astype(F32)
    dom_ref[...] = heads
    idx_ref[...] = jnp.broadcast_to(idx2d, idx_ref.shape)
    probs_ref[...] = jnp.pad(onehot, ((0, 0), (0, 128 - N_PART)))
    hh = hh_ref[...]
    s1 = lax.dot_general(onehot, hh, (((0,), (0,)), ((), ())),
                         preferred_element_type=F32, precision=HI)
    s2 = lax.dot_general(onehot, hh * hh, (((0,), (0,)), ((), ())),
                         preferred_element_type=F32, precision=HI)
    cnt = jnp.sum(onehot, axis=0, keepdims=True)

    @pl.when(pl.program_id(0) == 0)
    def _():
        s1_ref[...] = jnp.zeros_like(s1_ref)
        s2_ref[...] = jnp.zeros_like(s2_ref)
        cnt_ref[...] = jnp.zeros_like(cnt_ref)

    s1_ref[...] += s1
    s2_ref[...] += s2
    cnt_ref[...] += jnp.broadcast_to(cnt.reshape(N_PART, 1), cnt_ref.shape)


def _k_part2(hh_ref, s1_ref, s2_ref, cnt_ref, pg_ref, pbb_ref, probs_ref,
             w2_ref, b2_ref, out_ref):
    cnt = jnp.maximum(cnt_ref[...], 1.0)
    mean = s1_ref[...] / cnt
    var = s2_ref[...] / cnt - mean * mean
    inv = lax.rsqrt(var + EPS)
    row = lax.broadcasted_iota(jnp.int32, (N_PART, PART_LAYER), 0)
    col = lax.broadcasted_iota(jnp.int32, (N_PART, PART_LAYER), 1) // HIDDEN
    sel = (row == col).astype(F32)
    scale_full = inv * pg_ref[...]
    scale = jnp.sum(scale_full * sel, axis=0, keepdims=True)
    shift = jnp.sum((pbb_ref[...] - mean * scale_full) * sel,
                    axis=0, keepdims=True)
    hn = jnp.maximum(hh_ref[...] * scale + shift, 0.0)
    probs = probs_ref[...]
    bt = hn.shape[0]
    acc = jnp.zeros((bt, 128), F32)
    for part in range(N_PART):
        blk = hn[:, part * HIDDEN:(part + 1) * HIDDEN]
        oo = _bdot(blk, w2_ref[part]) + b2_ref[part:part + 1, :]
        acc += probs[:, part:part + 1] * oo
    out_ref[...] = acc


def _band(conv_w, wdim, cin, cout):
    mats = []
    for i in range(3):
        m = jnp.zeros(((wdim + 2) * cin, wdim * cout), F32)
        for j in range(3):
            m = m + jnp.kron(jnp.eye(wdim + 2, wdim, -j, dtype=F32),
                             conv_w[:, :, i, j].T)
        mats.append(m)
    return jnp.stack(mats).astype(BF16)


def _band_pooled(conv_w, h, cin, cout):
    hh = h // 2
    mats = []
    for i in range(3):
        m = jnp.zeros(((h + 4) * cin, hh * cout), F32)
        for j in range(3):
            sel = jnp.zeros((h + 4, hh), F32).at[
                2 * jnp.arange(hh) + 2 * j, jnp.arange(hh)].set(1.0)
            m = m + jnp.kron(sel, conv_w[:, :, i, j].T)
        mats.append(m)
    return jnp.stack(mats).astype(BF16)


def kernel(input_data, params, u):
    p = params
    B = input_data.shape[0]
    x2d = jnp.transpose(input_data, (0, 2, 3, 1)).reshape(B * 32, 96)

    w1 = _band(p['conv1_w'], 32, 3, 16)
    b1 = jnp.tile(p['conv1_b'], 32).reshape(1, 512)
    w2 = _band_pooled(p['conv2_w'], 32, 16, 32)
    b2 = jnp.tile(p['conv2_b'], 16).reshape(1, 512)
    w3 = _band_pooled(p['conv3_w'], 16, 32, 64)
    b3 = jnp.tile(p['conv3_b'], 8).reshape(1, 512)
    g1t = jnp.tile(p['bn1_g'], 32).reshape(1, 512)
    b1t = jnp.tile(p['bn1_b'], 32).reshape(1, 512)
    g2t = jnp.tile(p['bn2_g'], 16).reshape(1, 512)
    b2t = jnp.tile(p['bn2_b'], 16).reshape(1, 512)
    g3t = jnp.tile(p['bn3_g'], 8).reshape(1, 512)
    b3t = jnp.tile(p['bn3_b'], 8).reshape(1, 512)
    w4 = p['pre_w'].reshape(PRE_OUT, 64, 8, 8).transpose(2, 3, 1, 0) \
        .reshape(8, 512, PRE_OUT).astype(BF16)
    wd = jnp.concatenate(
        [p['disc_w'].T, p['pw1'].reshape(PART_LAYER, PRE_OUT).T],
        axis=1).astype(BF16)
    bd = jnp.concatenate([p['disc_b'], p['pb1'].reshape(-1)]).reshape(1, -1)
    wh = jnp.pad(jnp.concatenate([p['dfc_w'].T, p['sw_w'].T], axis=1),
                 ((0, 0), (0, 128 - NUM_DOM - N_PART))).astype(BF16)
    bh = jnp.pad(jnp.concatenate([p['dfc_b'], p['sw_b']]),
                 (0, 128 - NUM_DOM - N_PART)).reshape(1, -1)
    w2p = jnp.pad(jnp.transpose(p['pw2'], (0, 2, 1)),
                  ((0, 0), (0, 0), (0, 128 - NUM_CLASSES))).astype(BF16)
    b2p = jnp.pad(p['pb2'], ((0, 0), (0, 128 - NUM_CLASSES)))
    pg = p['pbn_g'].reshape(1, PART_LAYER)
    pbb = jnp.broadcast_to(p['pbn_b'].reshape(-1)[None, :],
                           (N_PART, PART_LAYER))

    full = lambda s: pl.BlockSpec(s, lambda i: tuple(0 for _ in s))

    bt1 = min(128, B)
    y1, st1 = pl.pallas_call(
        functools.partial(_k_conv1, bt=bt1),
        grid=(B // bt1,),
        in_specs=[pl.BlockSpec((bt1 * 32, 96), lambda i: (i, 0)),
                  full((3, 102, 512)), full((1, 512))],
        out_specs=[pl.BlockSpec((bt1 * 32, 512), lambda i: (i, 0)),
                   full((2, 512))],
        out_shape=[jax.ShapeDtypeStruct((B * 32, 512), F32),
                   jax.ShapeDtypeStruct((2, 512), F32)],
    )(x2d, w1, b1)

    bt2 = min(128, B)
    y2, st2 = pl.pallas_call(
        functools.partial(_k_norm_pool_conv, bt=bt2, n=float(B * 1024),
                          h=32, cin=16),
        grid=(B // bt2,),
        in_specs=[pl.BlockSpec((bt2 * 32, 512), lambda i: (i, 0)),
                  full((2, 512)), full((1, 512)), full((1, 512)),
                  full((3, 576, 512)), full((1, 512))],
        out_specs=[pl.BlockSpec((bt2 * 16, 512), lambda i: (i, 0)),
                   full((2, 512))],
        out_shape=[jax.ShapeDtypeStruct((B * 16, 512), F32),
                   jax.ShapeDtypeStruct((2, 512), F32)],
    )(y1, st1, g1t, b1t, w2, b2)

    bt3 = min(128, B)
    y3, st3 = pl.pallas_call(
        functools.partial(_k_norm_pool_conv, bt=bt3, n=float(B * 256),
                          h=16, cin=32),
        grid=(B // bt3,),
        in_specs=[pl.BlockSpec((bt3 * 16, 512), lambda i: (i, 0)),
                  full((2, 512)), full((1, 512)), full((1, 512)),
                  full((3, 640, 512)), full((1, 512))],
        out_specs=[pl.BlockSpec((bt3 * 8, 512), lambda i: (i, 0)),
                   full((2, 512))],
        out_shape=[jax.ShapeDtypeStruct((B * 8, 512), F32),
                   jax.ShapeDtypeStruct((2, 512), F32)],
    )(y2, st2, g2t, b2t, w3, b3)

    bt4 = min(256, B)
    dz, dst = pl.pallas_call(
        functools.partial(_k_pre_disc, bt=bt4, n=float(B * 64)),
        grid=(B // bt4,),
        in_specs=[pl.BlockSpec((bt4 * 8, 512), lambda i: (i, 0)),
                  full((2, 512)), full((1, 512)), full((1, 512)),
                  full((8, 512, PRE_OUT)), full((1, PRE_OUT)),
                  full((1, PRE_OUT)), full((1, PRE_OUT)),
                  full((PRE_OUT, 1024)), full((1, 1024))],
        out_specs=[pl.BlockSpec((bt4, 1024), lambda i: (i, 0)),
                   full((2, PART_LAYER))],
        out_shape=[jax.ShapeDtypeStruct((B, 1024), F32),
                   jax.ShapeDtypeStruct((2, PART_LAYER), F32)],
    )(y3, st3, g3t, b3t,
      w4, p['pre_b'].reshape(1, -1), p['ln_g'].reshape(1, -1),
      p['ln_b'].reshape(1, -1), wd, bd)

    bt6 = min(512, B)
    dom, idxp, probsp, s1, s2, cnt = pl.pallas_call(
        functools.partial(_k_route, n=float(B)),
        grid=(B // bt6,),
        in_specs=[pl.BlockSpec((bt6, PART_LAYER), lambda i: (i, 0)),
                  pl.BlockSpec((bt6, PART_LAYER), lambda i: (i, 1)),
                  full((2, PART_LAYER)),
                  full((1, PART_LAYER)), full((1, PART_LAYER)),
                  full((PART_LAYER, 128)), full((1, 128)),
                  pl.BlockSpec((bt6, N_PART), lambda i: (i, 0))],
        out_specs=[pl.BlockSpec((bt6, 128), lambda i: (i, 0)),
                   pl.BlockSpec((bt6, 128), lambda i: (i, 0)),
                   pl.BlockSpec((bt6, 128), lambda i: (i, 0)),
                   full((N_PART, PART_LAYER)), full((N_PART, PART_LAYER)),
                   full((N_PART, PART_LAYER))],
        out_shape=[jax.ShapeDtypeStruct((B, 128), F32),
                   jax.ShapeDtypeStruct((B, 128), jnp.int32),
                   jax.ShapeDtypeStruct((B, 128), F32),
                   jax.ShapeDtypeStruct((N_PART, PART_LAYER), F32),
                   jax.ShapeDtypeStruct((N_PART, PART_LAYER), F32),
                   jax.ShapeDtypeStruct((N_PART, PART_LAYER), F32)],
    )(dz, dz, dst, p['dbn_g'].reshape(1, -1), p['dbn_b'].reshape(1, -1),
      wh, bh, u)

    bt7 = min(512, B)
    outp = pl.pallas_call(
        _k_part2,
        grid=(B // bt7,),
        in_specs=[pl.BlockSpec((bt7, PART_LAYER), lambda i: (i, 1)),
                  full((N_PART, PART_LAYER)), full((N_PART, PART_LAYER)),
                  full((N_PART, PART_LAYER)),
                  full((1, PART_LAYER)), full((N_PART, PART_LAYER)),
                  pl.BlockSpec((bt7, 128), lambda i: (i, 0)),
                  full((N_PART, HIDDEN, 128)), full((N_PART, 128))],
        out_specs=pl.BlockSpec((bt7, 128), lambda i: (i, 0)),
        out_shape=jax.ShapeDtypeStruct((B, 128), F32),
    )(dz, s1, s2, cnt, pg, pbb, probsp, w2p, b2p)

    out = outp[:, :NUM_CLASSES]
    domain_out = dom[:, :NUM_DOM]
    idx = idxp[:, 0]
    probs = probsp[:, :N_PART]
    return out, domain_out, idx, probs
